# baseline (speedup 1.0000x reference)
import functools

import jax
import jax.numpy as jnp
import numpy as _np
from jax import lax
from jax.experimental import pallas as pl
from jax.experimental.pallas import tpu as pltpu
from jax.experimental.pallas import tpu_sc as plsc

N = 10000
E = 320000
D = 128
DH = 64
DW = 32
H = 256
NC, NS = 2, 16
NW = NC * NS
CHUNK = 128
NCHUNK = 80
EPW = NCHUNK * CHUNK
EPAD = EPW * NW
NPAD = 10240
STRIPE = NPAD // NS
DUMP = N
NSLOT = 8

_perm_np = _np.empty((D,), _np.int32)
for _h in range(2):
    for _c in range(DH):
        _f = 2 * _c if _c < DW else 2 * (_c - DW) + 1
        _perm_np[_h * DH + _c] = _h * DH + _f
_sigma_np = _np.argsort(_perm_np)

_mesh = plsc.VectorSubcoreMesh(core_axis_name="c", subcore_axis_name="s")


@functools.partial(
    pl.kernel,
    mesh=_mesh,
    compiler_params=pltpu.CompilerParams(use_tc_tiling_on_sc=False),
    out_type=jax.ShapeDtypeStruct((NC, 2, NPAD, 16), jnp.float32),
    scratch_types=[
        pltpu.VMEM((NCHUNK, CHUNK), jnp.int32),
        pltpu.VMEM((NCHUNK, CHUNK), jnp.int32),
        pltpu.VMEM((CHUNK, 16), jnp.float32),
        pltpu.VMEM_SHARED((NPAD, 16), jnp.float32),
        pltpu.VMEM_SHARED((NPAD, 16), jnp.float32),
        pltpu.SemaphoreType.DMA,
    ],
)
def _deg_kernel(src_hbm, dst_hbm, ones_hbm, zeros_hbm, out_hbm,
                src_v, dst_v, ones_v, dsrc_sh, ddst_sh, sem):
    c = lax.axis_index("c")
    s = lax.axis_index("s")
    wid = c * NS + s
    pltpu.sync_copy(src_hbm.at[wid], src_v)
    pltpu.sync_copy(dst_hbm.at[wid], dst_v)
    pltpu.sync_copy(ones_hbm, ones_v)
    pltpu.sync_copy(zeros_hbm, dsrc_sh.at[pl.ds(s * STRIPE, STRIPE)])
    pltpu.sync_copy(zeros_hbm, ddst_sh.at[pl.ds(s * STRIPE, STRIPE)])
    plsc.subcore_barrier()

    def fire(j, carry):
        pltpu.async_copy(ones_v, dsrc_sh.at[src_v.at[j]], sem, add=True)
        pltpu.async_copy(ones_v, ddst_sh.at[dst_v.at[j]], sem, add=True)
        return carry

    def drain(j, carry):
        pltpu.make_async_copy(ones_v, dsrc_sh.at[src_v.at[0]], sem).wait()
        pltpu.make_async_copy(ones_v, ddst_sh.at[dst_v.at[0]], sem).wait()
        return carry

    lax.fori_loop(0, NCHUNK, fire, 0)
    lax.fori_loop(0, NCHUNK, drain, 0)
    plsc.subcore_barrier()
    pltpu.sync_copy(dsrc_sh.at[pl.ds(s * STRIPE, STRIPE)],
                    out_hbm.at[c, 0, pl.ds(s * STRIPE, STRIPE)])
    pltpu.sync_copy(ddst_sh.at[pl.ds(s * STRIPE, STRIPE)],
                    out_hbm.at[c, 1, pl.ds(s * STRIPE, STRIPE)])


@functools.partial(
    pl.kernel,
    mesh=_mesh,
    compiler_params=pltpu.CompilerParams(use_tc_tiling_on_sc=False,
                                         needs_layout_passes=False),
    out_type=jax.ShapeDtypeStruct((2, NC, NPAD, DH), jnp.float32),
    scratch_types=[
        pltpu.VMEM((NCHUNK, CHUNK), jnp.int32),
        pltpu.VMEM((NCHUNK, CHUNK), jnp.int32),
        pltpu.VMEM((NSLOT, CHUNK, DH), jnp.bfloat16),
        pltpu.VMEM((4, CHUNK, DH), jnp.float32),
        pltpu.VMEM_SHARED((NPAD, DH), jnp.float32),
        [pltpu.SemaphoreType.DMA] * NSLOT,
        [pltpu.SemaphoreType.DMA] * 4,
    ],
)
def _agg_kernel(src_hbm, dst_hbm, table0_hbm, table1_hbm, zeros_hbm, out_hbm,
                src_v, dst_v, bf_v, f32_v, acc_sh, gsem, ssem):
    c = lax.axis_index("c")
    s = lax.axis_index("s")
    wid = c * NS + s
    pltpu.sync_copy(src_hbm.at[wid], src_v)
    pltpu.sync_copy(dst_hbm.at[wid], dst_v)

    def convert(b, fb):
        def row(r, carry):
            w0 = bf_v[b, r, pl.ds(0, 32)]
            w1 = bf_v[b, r, pl.ds(32, 32)]
            a0, b0 = plsc.unpack(w0, format=plsc.PackFormat.INTERLEAVED)
            a1, b1 = plsc.unpack(w1, format=plsc.PackFormat.INTERLEAVED)
            f32_v[fb, r, pl.ds(0, 16)] = a0
            f32_v[fb, r, pl.ds(32, 16)] = b0
            f32_v[fb, r, pl.ds(16, 16)] = a1
            f32_v[fb, r, pl.ds(48, 16)] = b1
            return carry

        lax.fori_loop(0, CHUNK, row, 0, unroll=4)

    for h, table_hbm in ((0, table0_hbm), (1, table1_hbm)):
        pltpu.sync_copy(zeros_hbm, acc_sh.at[pl.ds(s * STRIPE, STRIPE)])
        plsc.subcore_barrier()

        for b in range(NSLOT):
            pltpu.async_copy(table_hbm.at[src_v.at[b]], bf_v.at[b], gsem[b])

        def body(g, carry):
            j0 = g * NSLOT
            for b in range(NSLOT):
                j = j0 + b
                fb = b % 4
                pltpu.make_async_copy(table_hbm.at[src_v.at[j]],
                                      bf_v.at[b], gsem[b]).wait()

                if b >= 4:
                    pltpu.make_async_copy(f32_v.at[fb],
                                          acc_sh.at[dst_v.at[j - 4]],
                                          ssem[fb]).wait()
                else:
                    @pl.when(g > 0)
                    def _(fb=fb, j=j):
                        pltpu.make_async_copy(f32_v.at[fb],
                                              acc_sh.at[dst_v.at[j - 4]],
                                              ssem[fb]).wait()

                convert(b, fb)
                pltpu.async_copy(f32_v.at[fb], acc_sh.at[dst_v.at[j]],
                                 ssem[fb], add=True)

                @pl.when(j + NSLOT < NCHUNK)
                def _(b=b, j=j):
                    pltpu.async_copy(table_hbm.at[src_v.at[j + NSLOT]],
                                     bf_v.at[b], gsem[b])
            return carry

        lax.fori_loop(0, NCHUNK // NSLOT, body, 0)
        for fb in range(4):
            pltpu.make_async_copy(f32_v.at[fb],
                                  acc_sh.at[dst_v.at[NCHUNK - 4 + fb]],
                                  ssem[fb]).wait()
        plsc.subcore_barrier()
        pltpu.sync_copy(acc_sh.at[pl.ds(s * STRIPE, STRIPE)],
                        out_hbm.at[h, c, pl.ds(s * STRIPE, STRIPE)])
        plsc.subcore_barrier()


_TCR = 256
_TCG = NPAD // _TCR


def _norm_from(deg_ref, table):
    d = deg_ref[0, table] + deg_ref[1, table]
    return lax.rsqrt(jnp.clip(d[:, 0:1], 1.0, None))


def _merge_agg(agg_ref):
    return jnp.concatenate(
        [agg_ref[0, 0] + agg_ref[0, 1], agg_ref[1, 0] + agg_ref[1, 1]],
        axis=1)


def _scale_body(x_ref, deg_ref, out0_ref, out1_ref):
    xs = x_ref[...] * _norm_from(deg_ref, 0)
    out0_ref[...] = xs[:, :DH]
    out1_ref[...] = xs[:, DH:]


def _mid_body(agg_ref, deg_ref, w1_ref, b1_ref, w2_ref, out0_ref, out1_ref):
    a = _merge_agg(agg_ref) * _norm_from(deg_ref, 1)
    h = jnp.dot(a, w1_ref[...], preferred_element_type=jnp.float32) + b1_ref[...]
    r = jnp.maximum(h, 0.0) * _norm_from(deg_ref, 0)
    p = jnp.dot(r, w2_ref[...], preferred_element_type=jnp.float32)
    out0_ref[...] = p[:, :DH]
    out1_ref[...] = p[:, DH:]


def _final_body(agg_ref, deg_ref, b2_ref, out_ref):
    a = _merge_agg(agg_ref) * _norm_from(deg_ref, 1)
    out_ref[...] = a + b2_ref[...]


def _deg_spec():
    return pl.BlockSpec((NC, 2, _TCR, 16), lambda i: (0, 0, i, 0))


def _agg_spec():
    return pl.BlockSpec((2, NC, _TCR, DH), lambda i: (0, 0, i, 0))


def _half_specs():
    return [pl.BlockSpec((_TCR, DH), lambda i: (i, 0)) for _ in range(2)]


def _half_shapes():
    return [jax.ShapeDtypeStruct((NPAD, DH), jnp.float32) for _ in range(2)]


def _tc_scale(x_pad, degs):
    return pl.pallas_call(
        _scale_body,
        grid=(_TCG,),
        in_specs=[pl.BlockSpec((_TCR, D), lambda i: (i, 0)), _deg_spec()],
        out_specs=_half_specs(),
        out_shape=_half_shapes(),
    )(x_pad, degs)


def _tc_mid(agg, degs, W1, b1, W2):
    return pl.pallas_call(
        _mid_body,
        grid=(_TCG,),
        in_specs=[
            _agg_spec(),
            _deg_spec(),
            pl.BlockSpec((D, H), lambda i: (0, 0)),
            pl.BlockSpec((1, H), lambda i: (0, 0)),
            pl.BlockSpec((H, D), lambda i: (0, 0)),
        ],
        out_specs=_half_specs(),
        out_shape=_half_shapes(),
    )(agg, degs, W1, b1, W2)


def _tc_final(agg, degs, b2):
    return pl.pallas_call(
        _final_body,
        grid=(_TCG,),
        in_specs=[
            _agg_spec(),
            _deg_spec(),
            pl.BlockSpec((1, D), lambda i: (0, 0)),
        ],
        out_specs=pl.BlockSpec((_TCR, D), lambda i: (i, 0)),
        out_shape=jax.ShapeDtypeStruct((NPAD, D), jnp.float32),
    )(agg, degs, b2)


def _pack_bf16(a):
    return a.astype(jnp.bfloat16)


def kernel(x, edge_index, W1, b1, W2, b2):
    src = edge_index[0].astype(jnp.int32)
    dst = edge_index[1].astype(jnp.int32)
    pad = jnp.full((EPAD - E,), DUMP, jnp.int32)
    srcp = jnp.concatenate([src, pad]).reshape(NW, NCHUNK, CHUNK)
    dstp = jnp.concatenate([dst, pad]).reshape(NW, NCHUNK, CHUNK)
    x_pad = jnp.zeros((NPAD, D), jnp.float32).at[:N].set(x)
    zeros_rows = jnp.zeros((STRIPE, DH), jnp.float32)
    zeros16 = jnp.zeros((STRIPE, 16), jnp.float32)
    ones16 = jnp.ones((CHUNK, 16), jnp.float32)
    W1p = W1[jnp.asarray(_perm_np), :]
    W2p = W2[:, jnp.asarray(_sigma_np)]

    degs = _deg_kernel(srcp, dstp, ones16, zeros16)
    xs0, xs1 = _tc_scale(x_pad, degs)
    agg1 = _agg_kernel(srcp, dstp, _pack_bf16(xs0), _pack_bf16(xs1),
                       zeros_rows)
    p0, p1 = _tc_mid(agg1, degs, W1p, b1.reshape(1, H), W2p)
    agg2 = _agg_kernel(srcp, dstp, _pack_bf16(p0), _pack_bf16(p1),
                       zeros_rows)
    out = _tc_final(agg2, degs, b2.reshape(1, D))
    return out[:N]


# speedup vs baseline: 1.0788x; 1.0788x over previous
import functools

import jax
import jax.numpy as jnp
import numpy as _np
from jax import lax
from jax.experimental import pallas as pl
from jax.experimental.pallas import tpu as pltpu
from jax.experimental.pallas import tpu_sc as plsc

N = 10000
E = 320000
D = 128
DH = 64
DW = 32
H = 256
NC, NS = 2, 16
NW = NC * NS
CHUNK = 128
NCHUNK = 80
EPW = NCHUNK * CHUNK
EPAD = EPW * NW
NPAD = 10240
STRIPE = NPAD // NS
DUMP = N
NSLOT = 8

_perm_np = _np.empty((D,), _np.int32)
for _h in range(2):
    for _c in range(DH):
        _f = 2 * _c if _c < DW else 2 * (_c - DW) + 1
        _perm_np[_h * DH + _c] = _h * DH + _f
_sigma_np = _np.argsort(_perm_np)

_mesh = plsc.VectorSubcoreMesh(core_axis_name="c", subcore_axis_name="s")


@functools.partial(
    pl.kernel,
    mesh=_mesh,
    compiler_params=pltpu.CompilerParams(use_tc_tiling_on_sc=False),
    out_type=jax.ShapeDtypeStruct((NC, 2, NPAD, 16), jnp.float32),
    scratch_types=[
        pltpu.VMEM((NCHUNK, CHUNK), jnp.int32),
        pltpu.VMEM((NCHUNK, CHUNK), jnp.int32),
        pltpu.VMEM((CHUNK, 16), jnp.float32),
        pltpu.VMEM_SHARED((NPAD, 16), jnp.float32),
        pltpu.VMEM_SHARED((NPAD, 16), jnp.float32),
        pltpu.SemaphoreType.DMA,
    ],
)
def _deg_kernel(src_hbm, dst_hbm, ones_hbm, zeros_hbm, out_hbm,
                src_v, dst_v, ones_v, dsrc_sh, ddst_sh, sem):
    c = lax.axis_index("c")
    s = lax.axis_index("s")
    wid = c * NS + s
    pltpu.sync_copy(src_hbm.at[wid], src_v)
    pltpu.sync_copy(dst_hbm.at[wid], dst_v)
    pltpu.sync_copy(ones_hbm, ones_v)
    pltpu.sync_copy(zeros_hbm, dsrc_sh.at[pl.ds(s * STRIPE, STRIPE)])
    pltpu.sync_copy(zeros_hbm, ddst_sh.at[pl.ds(s * STRIPE, STRIPE)])
    plsc.subcore_barrier()

    def fire(j, carry):
        pltpu.async_copy(ones_v, dsrc_sh.at[src_v.at[j]], sem, add=True)
        pltpu.async_copy(ones_v, ddst_sh.at[dst_v.at[j]], sem, add=True)
        return carry

    def drain(j, carry):
        pltpu.make_async_copy(ones_v, dsrc_sh.at[src_v.at[0]], sem).wait()
        pltpu.make_async_copy(ones_v, ddst_sh.at[dst_v.at[0]], sem).wait()
        return carry

    lax.fori_loop(0, NCHUNK, fire, 0)
    lax.fori_loop(0, NCHUNK, drain, 0)
    plsc.subcore_barrier()
    pltpu.sync_copy(dsrc_sh.at[pl.ds(s * STRIPE, STRIPE)],
                    out_hbm.at[c, 0, pl.ds(s * STRIPE, STRIPE)])
    pltpu.sync_copy(ddst_sh.at[pl.ds(s * STRIPE, STRIPE)],
                    out_hbm.at[c, 1, pl.ds(s * STRIPE, STRIPE)])


@functools.partial(
    pl.kernel,
    mesh=_mesh,
    compiler_params=pltpu.CompilerParams(use_tc_tiling_on_sc=False,
                                         needs_layout_passes=False),
    out_type=jax.ShapeDtypeStruct((2, NC, NPAD, DH), jnp.float32),
    scratch_types=[
        pltpu.VMEM((NCHUNK, CHUNK), jnp.int32),
        pltpu.VMEM((NCHUNK, CHUNK), jnp.int32),
        pltpu.VMEM((NSLOT, CHUNK, DH), jnp.bfloat16),
        pltpu.VMEM((4, CHUNK, DH), jnp.float32),
        pltpu.VMEM_SHARED((NPAD, DH), jnp.float32),
        [pltpu.SemaphoreType.DMA] * NSLOT,
        [pltpu.SemaphoreType.DMA] * 4,
    ],
)
def _agg_kernel(src_hbm, dst_hbm, table0_hbm, table1_hbm, zeros_hbm, out_hbm,
                src_v, dst_v, bf_v, f32_v, acc_sh, gsem, ssem):
    c = lax.axis_index("c")
    s = lax.axis_index("s")
    wid = c * NS + s
    pltpu.sync_copy(src_hbm.at[wid], src_v)
    pltpu.sync_copy(dst_hbm.at[wid], dst_v)

    def convert(b, fb):
        def row(r, carry):
            w0 = bf_v[b, r, pl.ds(0, 32)]
            w1 = bf_v[b, r, pl.ds(32, 32)]
            a0, b0 = plsc.unpack(w0, format=plsc.PackFormat.INTERLEAVED)
            a1, b1 = plsc.unpack(w1, format=plsc.PackFormat.INTERLEAVED)
            f32_v[fb, r, pl.ds(0, 16)] = a0
            f32_v[fb, r, pl.ds(32, 16)] = b0
            f32_v[fb, r, pl.ds(16, 16)] = a1
            f32_v[fb, r, pl.ds(48, 16)] = b1
            return carry

        lax.fori_loop(0, CHUNK, row, 0)

    for h, table_hbm in ((0, table0_hbm), (1, table1_hbm)):
        pltpu.sync_copy(zeros_hbm, acc_sh.at[pl.ds(s * STRIPE, STRIPE)])
        plsc.subcore_barrier()

        for b in range(NSLOT):
            pltpu.async_copy(table_hbm.at[src_v.at[b]], bf_v.at[b], gsem[b])

        def body(g, carry):
            j0 = g * NSLOT
            for b in range(NSLOT):
                j = j0 + b
                fb = b % 4
                pltpu.make_async_copy(table_hbm.at[src_v.at[j]],
                                      bf_v.at[b], gsem[b]).wait()

                if b >= 4:
                    pltpu.make_async_copy(f32_v.at[fb],
                                          acc_sh.at[dst_v.at[j - 4]],
                                          ssem[fb]).wait()
                else:
                    @pl.when(g > 0)
                    def _(fb=fb, j=j):
                        pltpu.make_async_copy(f32_v.at[fb],
                                              acc_sh.at[dst_v.at[j - 4]],
                                              ssem[fb]).wait()

                convert(b, fb)
                pltpu.async_copy(f32_v.at[fb], acc_sh.at[dst_v.at[j]],
                                 ssem[fb], add=True)

                @pl.when(j + NSLOT < NCHUNK)
                def _(b=b, j=j):
                    pltpu.async_copy(table_hbm.at[src_v.at[j + NSLOT]],
                                     bf_v.at[b], gsem[b])
            return carry

        lax.fori_loop(0, NCHUNK // NSLOT, body, 0)
        for fb in range(4):
            pltpu.make_async_copy(f32_v.at[fb],
                                  acc_sh.at[dst_v.at[NCHUNK - 4 + fb]],
                                  ssem[fb]).wait()
        plsc.subcore_barrier()
        pltpu.sync_copy(acc_sh.at[pl.ds(s * STRIPE, STRIPE)],
                        out_hbm.at[h, c, pl.ds(s * STRIPE, STRIPE)])
        plsc.subcore_barrier()


_TCR = 256
_TCG = NPAD // _TCR


def _norm_from(deg_ref, table):
    d = deg_ref[0, table] + deg_ref[1, table]
    return lax.rsqrt(jnp.clip(d[:, 0:1], 1.0, None))


def _merge_agg(agg_ref):
    return jnp.concatenate(
        [agg_ref[0, 0] + agg_ref[0, 1], agg_ref[1, 0] + agg_ref[1, 1]],
        axis=1)


def _scale_body(x_ref, deg_ref, out0_ref, out1_ref):
    xs = x_ref[...] * _norm_from(deg_ref, 0)
    out0_ref[...] = xs[:, :DH]
    out1_ref[...] = xs[:, DH:]


def _mid_body(agg_ref, deg_ref, w1_ref, b1_ref, w2_ref, out0_ref, out1_ref):
    a = _merge_agg(agg_ref) * _norm_from(deg_ref, 1)
    h = jnp.dot(a, w1_ref[...], preferred_element_type=jnp.float32) + b1_ref[...]
    r = jnp.maximum(h, 0.0) * _norm_from(deg_ref, 0)
    p = jnp.dot(r, w2_ref[...], preferred_element_type=jnp.float32)
    out0_ref[...] = p[:, :DH]
    out1_ref[...] = p[:, DH:]


def _final_body(agg_ref, deg_ref, b2_ref, out_ref):
    a = _merge_agg(agg_ref) * _norm_from(deg_ref, 1)
    out_ref[...] = a + b2_ref[...]


def _deg_spec():
    return pl.BlockSpec((NC, 2, _TCR, 16), lambda i: (0, 0, i, 0))


def _agg_spec():
    return pl.BlockSpec((2, NC, _TCR, DH), lambda i: (0, 0, i, 0))


def _half_specs():
    return [pl.BlockSpec((_TCR, DH), lambda i: (i, 0)) for _ in range(2)]


def _half_shapes():
    return [jax.ShapeDtypeStruct((NPAD, DH), jnp.float32) for _ in range(2)]


def _tc_scale(x_pad, degs):
    return pl.pallas_call(
        _scale_body,
        grid=(_TCG,),
        in_specs=[pl.BlockSpec((_TCR, D), lambda i: (i, 0)), _deg_spec()],
        out_specs=_half_specs(),
        out_shape=_half_shapes(),
    )(x_pad, degs)


def _tc_mid(agg, degs, W1, b1, W2):
    return pl.pallas_call(
        _mid_body,
        grid=(_TCG,),
        in_specs=[
            _agg_spec(),
            _deg_spec(),
            pl.BlockSpec((D, H), lambda i: (0, 0)),
            pl.BlockSpec((1, H), lambda i: (0, 0)),
            pl.BlockSpec((H, D), lambda i: (0, 0)),
        ],
        out_specs=_half_specs(),
        out_shape=_half_shapes(),
    )(agg, degs, W1, b1, W2)


def _tc_final(agg, degs, b2):
    return pl.pallas_call(
        _final_body,
        grid=(_TCG,),
        in_specs=[
            _agg_spec(),
            _deg_spec(),
            pl.BlockSpec((1, D), lambda i: (0, 0)),
        ],
        out_specs=pl.BlockSpec((_TCR, D), lambda i: (i, 0)),
        out_shape=jax.ShapeDtypeStruct((NPAD, D), jnp.float32),
    )(agg, degs, b2)


def _pack_bf16(a):
    return a.astype(jnp.bfloat16)


def kernel(x, edge_index, W1, b1, W2, b2):
    src = edge_index[0].astype(jnp.int32)
    dst = edge_index[1].astype(jnp.int32)
    pad = jnp.full((EPAD - E,), DUMP, jnp.int32)
    srcp = jnp.concatenate([src, pad]).reshape(NW, NCHUNK, CHUNK)
    dstp = jnp.concatenate([dst, pad]).reshape(NW, NCHUNK, CHUNK)
    x_pad = jnp.zeros((NPAD, D), jnp.float32).at[:N].set(x)
    zeros_rows = jnp.zeros((STRIPE, DH), jnp.float32)
    zeros16 = jnp.zeros((STRIPE, 16), jnp.float32)
    ones16 = jnp.ones((CHUNK, 16), jnp.float32)
    W1p = W1[jnp.asarray(_perm_np), :]
    W2p = W2[:, jnp.asarray(_sigma_np)]

    degs = _deg_kernel(srcp, dstp, ones16, zeros16)
    deg_out = degs[0, 0, :, 0] + degs[1, 0, :, 0]
    deg_in = degs[0, 1, :, 0] + degs[1, 1, :, 0]
    norm_out = lax.rsqrt(jnp.clip(deg_out, 1.0, None))[:, None]
    norm_in = lax.rsqrt(jnp.clip(deg_in, 1.0, None))[:, None]
    xs = x_pad * norm_out
    agg1 = _agg_kernel(srcp, dstp, _pack_bf16(xs[:, :DH]),
                       _pack_bf16(xs[:, DH:]), zeros_rows)
    p0, p1 = _tc_mid(agg1, degs, W1p, b1.reshape(1, H), W2p)
    agg2 = _agg_kernel(srcp, dstp, _pack_bf16(p0), _pack_bf16(p1),
                       zeros_rows)
    a2 = (agg2[0, 0] + agg2[0, 1]).astype(jnp.float32)
    b2p = (agg2[1, 0] + agg2[1, 1]).astype(jnp.float32)
    out = jnp.concatenate([a2, b2p], axis=1) * norm_in + b2[None, :]
    return out[:N]
